# baseline (speedup 1.0000x reference)
import functools
import math

import jax
import jax.numpy as jnp
from jax import lax
from jax.experimental import pallas as pl
from jax.experimental.pallas import tpu as pltpu
from jax.experimental.pallas import tpu_sc as plsc

VOCAB = 1000000
D = 32
B = 4096
L = 50

NC = 2
NS = 16
NW = NC * NS
LANES = 16

N_TOTAL = B * L
B_PER_W = N_TOTAL // NW
CHUNK = 128
N_CHUNKS = B_PER_W // CHUNK

SCALE = math.sqrt(D)


@functools.partial(
    pl.kernel,
    out_type=jax.ShapeDtypeStruct((NW, B_PER_W, D), jnp.float32),
    mesh=plsc.VectorSubcoreMesh(core_axis_name="c", subcore_axis_name="s"),
    scratch_types=[
        pltpu.VMEM((N_CHUNKS, CHUNK), jnp.int32),
        pltpu.VMEM((CHUNK, D), jnp.float32),
        pltpu.SemaphoreType.DMA,
    ],
    compiler_params=pltpu.CompilerParams(use_tc_tiling_on_sc=False),
)
def _emb_lookup(x_hbm, table_hbm, out_hbm, idx_v, rows_v, gsem):
    wid = lax.axis_index("s") * NC + lax.axis_index("c")
    pltpu.sync_copy(x_hbm.at[wid], idx_v)

    @pl.loop(0, N_CHUNKS)
    def _chunk(c):
        pltpu.async_copy(table_hbm.at[idx_v.at[c]], rows_v, gsem).wait()

        @pl.loop(0, CHUNK)
        def _row(r):
            for h in range(D // LANES):
                sl = pl.ds(h * LANES, LANES)
                rows_v[r, sl] = rows_v[r, sl] * SCALE

        pltpu.sync_copy(rows_v, out_hbm.at[wid, pl.ds(c * CHUNK, CHUNK)])


def kernel(x, table):
    xf = x.reshape(NW, N_CHUNKS, CHUNK).astype(jnp.int32)
    out = _emb_lookup(xf, table)
    return out.reshape(B, L, D)


# speedup vs baseline: 1.0853x; 1.0853x over previous
import functools
import math

import jax
import jax.numpy as jnp
from jax import lax
from jax.experimental import pallas as pl
from jax.experimental.pallas import tpu as pltpu
from jax.experimental.pallas import tpu_sc as plsc

VOCAB = 1000000
D = 32
B = 4096
L = 50

NC = 2
NS = 16
NW = NC * NS
LANES = 16

N_TOTAL = B * L
B_PER_W = N_TOTAL // NW
CHUNK = 128
N_CHUNKS = B_PER_W // CHUNK
NBUF = 5
N_GROUPS = N_CHUNKS // NBUF

SCALE = math.sqrt(D)


@functools.partial(
    pl.kernel,
    out_type=jax.ShapeDtypeStruct((NW, B_PER_W, D), jnp.float32),
    mesh=plsc.VectorSubcoreMesh(core_axis_name="c", subcore_axis_name="s"),
    scratch_types=[
        pltpu.VMEM((N_CHUNKS, CHUNK), jnp.int32),
        *[pltpu.VMEM((CHUNK, D), jnp.float32) for _ in range(NBUF)],
        *[pltpu.SemaphoreType.DMA for _ in range(NBUF)],
    ],
    compiler_params=pltpu.CompilerParams(use_tc_tiling_on_sc=False),
)
def _emb_lookup(x_hbm, table_hbm, out_hbm, idx_v, *bufs_and_sems):
    rows = bufs_and_sems[:NBUF]
    gsem = bufs_and_sems[NBUF:]
    wid = lax.axis_index("s") * NC + lax.axis_index("c")
    pltpu.sync_copy(x_hbm.at[wid], idx_v)

    for b in range(NBUF):
        pltpu.async_copy(table_hbm.at[idx_v.at[b]], rows[b], gsem[b])

    @pl.loop(0, N_GROUPS)
    def _group(g):
        for b in range(NBUF):
            c = g * NBUF + b
            pltpu.make_async_copy(
                table_hbm.at[idx_v.at[c]], rows[b], gsem[b]).wait()

            @pl.loop(0, CHUNK, unroll=8)
            def _row(r):
                for h in range(D // LANES):
                    sl = pl.ds(h * LANES, LANES)
                    rows[b][r, sl] = rows[b][r, sl] * SCALE

            pltpu.sync_copy(rows[b], out_hbm.at[wid, pl.ds(c * CHUNK, CHUNK)])

            @pl.when(g + 1 < N_GROUPS)
            def _prefetch():
                pltpu.async_copy(
                    table_hbm.at[idx_v.at[c + NBUF]], rows[b], gsem[b])


def kernel(x, table):
    xf = x.reshape(NW, N_CHUNKS, CHUNK).astype(jnp.int32)
    out = _emb_lookup(xf, table)
    return out.reshape(B, L, D)
